# baseline (speedup 1.0000x reference)
import jax
import jax.numpy as jnp
from jax import lax
from jax.experimental import pallas as pl
from jax.experimental.pallas import tpu as pltpu
from jax.experimental.pallas import tpu_sc as plsc

D_MODEL = 64
SCALE = 8.0
CHUNK = 128
NUM_CORES = 2
NUM_SUBCORES = 16
NUM_WORKERS = NUM_CORES * NUM_SUBCORES
LANES = 16


def kernel(x, table):
    b, s = x.shape
    n = b * s
    assert n % (NUM_WORKERS * CHUNK) == 0
    per_worker = n // NUM_WORKERS
    n_chunks = per_worker // CHUNK
    idx = x.reshape(n)
    vocab, d = table.shape
    table_pairs = table.reshape(vocab // 2, 2 * d)

    mesh = plsc.VectorSubcoreMesh(core_axis_name="c", subcore_axis_name="s")

    @jax.jit
    @pl.kernel(
        out_type=jax.ShapeDtypeStruct((n, D_MODEL), table.dtype),
        mesh=mesh,
        scratch_types=[
            pltpu.VMEM((CHUNK,), jnp.int32),
            pltpu.VMEM((CHUNK,), jnp.int32),
            pltpu.VMEM((CHUNK, 2 * D_MODEL), jnp.float32),
            pltpu.VMEM((CHUNK, D_MODEL), jnp.float32),
            pltpu.SemaphoreType.DMA,
        ],
    )
    def gather_scale(table_hbm, idx_hbm, out_hbm, idx_v, row_v, g_v, o_v, sem):
        wid = lax.axis_index("s") * NUM_CORES + lax.axis_index("c")
        base = wid * per_worker

        @pl.loop(0, n_chunks)
        def _(ci):
            off = base + ci * CHUNK
            pltpu.sync_copy(idx_hbm.at[pl.ds(off, CHUNK)], idx_v)

            @pl.loop(0, CHUNK, step=LANES)
            def _(k):
                slc = pl.ds(k, LANES)
                row_v.at[slc][...] = idx_v.at[slc][...] >> 1

            pltpu.async_copy(table_hbm.at[row_v], g_v, sem).wait()

            @pl.loop(0, CHUNK)
            def _(r):
                half = (idx_v[pl.ds(r, 1)][0] & 1) * D_MODEL
                for c in range(0, D_MODEL, LANES):
                    o_v.at[pl.ds(r, 1), pl.ds(c, LANES)][...] = (
                        g_v.at[pl.ds(r, 1), pl.ds(half + c, LANES)][...]
                        * SCALE
                    )

            pltpu.sync_copy(o_v, out_hbm.at[pl.ds(off, CHUNK)])

    out = gather_scale(table_pairs, idx)
    return out.reshape(b, s, D_MODEL)


# speedup vs baseline: 1.2828x; 1.2828x over previous
import jax
import jax.numpy as jnp
from jax import lax
from jax.experimental import pallas as pl
from jax.experimental.pallas import tpu as pltpu
from jax.experimental.pallas import tpu_sc as plsc

D_MODEL = 64
SCALE = 8.0
CHUNK = 128
NUM_CORES = 2
NUM_SUBCORES = 16
NUM_WORKERS = NUM_CORES * NUM_SUBCORES
LANES = 16


def kernel(x, table):
    b, s = x.shape
    n = b * s
    assert n % (NUM_WORKERS * 2 * CHUNK) == 0
    per_worker = n // NUM_WORKERS
    n_chunks = per_worker // CHUNK
    idx = x.reshape(n)
    vocab, d = table.shape
    table_pairs = table.reshape(vocab // 2, 2 * d)

    mesh = plsc.VectorSubcoreMesh(core_axis_name="c", subcore_axis_name="s")

    @jax.jit
    @pl.kernel(
        out_type=jax.ShapeDtypeStruct((n, D_MODEL), table.dtype),
        mesh=mesh,
        scratch_types=[
            pltpu.VMEM((per_worker,), jnp.int32),
            pltpu.VMEM((per_worker,), jnp.int32),
            pltpu.VMEM((CHUNK, 2 * D_MODEL), jnp.float32),
            pltpu.VMEM((CHUNK, 2 * D_MODEL), jnp.float32),
            pltpu.VMEM((CHUNK, D_MODEL), jnp.float32),
            pltpu.VMEM((CHUNK, D_MODEL), jnp.float32),
            pltpu.SemaphoreType.DMA,
            pltpu.SemaphoreType.DMA,
            pltpu.SemaphoreType.DMA,
            pltpu.SemaphoreType.DMA,
            pltpu.SemaphoreType.DMA,
        ],
    )
    def gather_scale(
        table_hbm, idx_hbm, out_hbm,
        half_all, row_all, g0, g1, o0, o1,
        sem_i, sg0, sg1, so0, so1,
    ):
        wid = lax.axis_index("s") * NUM_CORES + lax.axis_index("c")
        base = wid * per_worker

        pltpu.async_copy(idx_hbm.at[pl.ds(base, per_worker)], half_all, sem_i).wait()

        @pl.loop(0, per_worker, step=LANES)
        def _(k):
            slc = pl.ds(k, LANES)
            v = half_all.at[slc][...]
            row_all.at[slc][...] = v >> 1
            half_all.at[slc][...] = (v & 1) * D_MODEL

        gbufs = (g0, g1)
        obufs = (o0, o1)
        gsems = (sg0, sg1)
        osems = (so0, so1)

        def gather_desc(ci, g, sg):
            return pltpu.make_async_copy(
                table_hbm.at[row_all.at[pl.ds(ci * CHUNK, CHUNK)]], g, sg
            )

        def out_desc(ci, o, so):
            return pltpu.make_async_copy(
                o, out_hbm.at[pl.ds(base + ci * CHUNK, CHUNK)], so
            )

        gather_desc(0, g0, sg0).start()
        gather_desc(1, g1, sg1).start()

        @pl.loop(0, n_chunks // 2)
        def _(ph):
            ci = 2 * ph
            for j in range(2):
                g, o, sg, so = gbufs[j], obufs[j], gsems[j], osems[j]
                cj = ci + j
                gather_desc(cj, g, sg).wait()

                @pl.when(ph > 0)
                def _():
                    out_desc(cj - 2, o, so).wait()

                @pl.loop(0, CHUNK)
                def _(r):
                    half = half_all[pl.ds(cj * CHUNK + r, 1)][0]
                    for c in range(0, D_MODEL, LANES):
                        o.at[pl.ds(r, 1), pl.ds(c, LANES)][...] = (
                            g.at[pl.ds(r, 1), pl.ds(half + c, LANES)][...]
                            * SCALE
                        )

                out_desc(cj, o, so).start()

                @pl.when(cj + 2 < n_chunks)
                def _():
                    gather_desc(cj + 2, g, sg).start()

        out_desc(n_chunks - 2, o0, so0).wait()
        out_desc(n_chunks - 1, o1, so1).wait()

    out = gather_scale(table_pairs, idx)
    return out.reshape(b, s, D_MODEL)


# speedup vs baseline: 1.9685x; 1.5345x over previous
import jax
import jax.numpy as jnp
from jax import lax
from jax.experimental import pallas as pl
from jax.experimental.pallas import tpu as pltpu
from jax.experimental.pallas import tpu_sc as plsc

D_MODEL = 64
SCALE = 8.0
CHUNK = 128
NUM_CORES = 2
NUM_SUBCORES = 16
NUM_WORKERS = NUM_CORES * NUM_SUBCORES
LANES = 16
NBUF = 4


def kernel(x, table):
    b, s = x.shape
    n = b * s
    assert n % (NUM_WORKERS * 2 * NBUF * CHUNK) == 0
    per_worker = n // NUM_WORKERS
    slab = per_worker // 2
    slab_chunks = slab // CHUNK
    idx = x.reshape(n)
    vocab, d = table.shape
    table_pairs = table.reshape(vocab // 2, 2 * d)

    mesh = plsc.VectorSubcoreMesh(core_axis_name="c", subcore_axis_name="s")

    @jax.jit
    @pl.kernel(
        out_type=jax.ShapeDtypeStruct((n, D_MODEL), table.dtype),
        mesh=mesh,
        scratch_types=(
            [pltpu.VMEM((slab,), jnp.int32)]
            + [pltpu.VMEM((CHUNK,), jnp.int32) for _ in range(NBUF)]
            + [pltpu.VMEM((NBUF * CHUNK, 2 * D_MODEL), jnp.float32)]
            + [pltpu.VMEM((CHUNK, D_MODEL), jnp.float32) for _ in range(2)]
            + [pltpu.SemaphoreType.DMA for _ in range(NBUF + 3)]
        ),
    )
    def gather_scale(table_hbm, idx_hbm, out_hbm, idx_all, *rest):
        rowbufs = rest[:NBUF]
        g_big = rest[NBUF]
        obufs = rest[NBUF + 1 : NBUF + 3]
        gsems = rest[NBUF + 3 : 2 * NBUF + 3]
        osems = rest[2 * NBUF + 3 : 2 * NBUF + 5]
        sem_i = rest[2 * NBUF + 5]

        wid = lax.axis_index("s") * NUM_CORES + lax.axis_index("c")
        base = wid * per_worker

        def run_phase(p):
            pbase = base + p * slab

            pltpu.async_copy(
                idx_hbm.at[pl.ds(pbase, slab)], idx_all, sem_i
            ).wait()

            def gather_desc(q):
                return pltpu.make_async_copy(
                    table_hbm.at[rowbufs[q]],
                    g_big.at[pl.ds(q * CHUNK, CHUNK)],
                    gsems[q],
                )

            def issue_gather(ci, q):
                @pl.loop(0, CHUNK, step=LANES)
                def _(k):
                    rowbufs[q].at[pl.ds(k, LANES)][...] = (
                        idx_all.at[pl.ds(ci * CHUNK + k, LANES)][...] >> 1
                    )

                gather_desc(q).start()

            def out_desc(ci, q):
                return pltpu.make_async_copy(
                    obufs[q],
                    out_hbm.at[pl.ds(pbase + ci * CHUNK, CHUNK)],
                    osems[q],
                )

            for q in range(NBUF):
                issue_gather(q, q)

            @pl.loop(0, slab_chunks // NBUF)
            def _(ph):
                ci = NBUF * ph
                for q in range(NBUF):
                    cj = ci + q
                    ob = q % 2
                    o = obufs[ob]
                    gather_desc(q).wait()

                    if q < 2:
                        @pl.when(ph > 0)
                        def _():
                            out_desc(cj - 2, ob).wait()
                    else:
                        out_desc(cj - 2, ob).wait()

                    zeros16 = jnp.zeros((LANES, 1), jnp.int32)
                    rot1 = jnp.minimum(
                        lax.iota(jnp.int32, LANES) + 1, LANES - 1
                    ).reshape(LANES, 1)
                    dnums = lax.GatherDimensionNumbers(
                        offset_dims=(),
                        collapsed_slice_dims=(0,),
                        start_index_map=(0,),
                    )

                    def bcast0(v):
                        return lax.gather(
                            v, zeros16, dnums, (1,),
                            mode=lax.GatherScatterMode.PROMISE_IN_BOUNDS,
                        )

                    def rot(v):
                        return lax.gather(
                            v, rot1, dnums, (1,),
                            mode=lax.GatherScatterMode.PROMISE_IN_BOUNDS,
                        )

                    @pl.loop(0, CHUNK, step=LANES)
                    def _(k):
                        vpar = (
                            idx_all[pl.ds(cj * CHUNK + k, LANES)] & 1
                        ).astype(jnp.float32) * SCALE
                        vcur = vpar
                        for t in range(LANES):
                            r = k + t
                            w8 = bcast0(vcur)
                            if t + 1 < LANES:
                                vcur = rot(vcur)
                            w8c = SCALE - w8
                            grow = g_big.at[q * CHUNK + r]
                            orow = o.at[r]
                            for c in range(0, D_MODEL, LANES):
                                left = grow.at[pl.ds(c, LANES)][...]
                                right = grow.at[pl.ds(D_MODEL + c, LANES)][...]
                                orow.at[pl.ds(c, LANES)][...] = (
                                    left * w8c + right * w8
                                )

                    out_desc(cj, ob).start()

                    @pl.when(cj + NBUF < slab_chunks)
                    def _():
                        issue_gather(cj + NBUF, q)

            out_desc(slab_chunks - 2, 0).wait()
            out_desc(slab_chunks - 1, 1).wait()

        for p in range(2):
            run_phase(p)

    out = gather_scale(table_pairs, idx)
    return out.reshape(b, s, D_MODEL)
